# baseline (speedup 1.0000x reference)
import functools

import jax
import jax.numpy as jnp
from jax import lax
from jax.experimental import pallas as pl
from jax.experimental.pallas import tpu as pltpu
from jax.experimental.pallas import tpu_sc as plsc

NENT = 100000
NREL = 64
STT = 256
ABSD = 128
REL0 = 128
RELL = 256
B = 1024
GAMMA = 12.0

_NC, _NS = 2, 16
_NW = _NC * _NS
_BW = B // _NW
_NT = 10


def _sc_gather_fn(idx_s_hbm, idx_o_hbm, e_hbm, *rest):
    tabs = (e_hbm,) + rest[:_NT - 1]
    outs = rest[_NT - 1:_NT - 1 + 2 * _NT]
    idx_s_v, idx_o_v, erows_v, arows_v, sem = rest[_NT - 1 + 2 * _NT:]

    wid = lax.axis_index("s") * _NC + lax.axis_index("c")
    base = wid * _BW
    pltpu.sync_copy(idx_s_hbm.at[pl.ds(base, _BW)], idx_s_v)
    pltpu.sync_copy(idx_o_hbm.at[pl.ds(base, _BW)], idx_o_v)

    for j in range(_NT):
        buf = erows_v if j == 0 else arows_v
        pltpu.async_copy(tabs[j].at[idx_s_v], buf, sem).wait()
        pltpu.sync_copy(buf, outs[2 * j].at[pl.ds(base, _BW)])
        pltpu.async_copy(tabs[j].at[idx_o_v], buf, sem).wait()
        pltpu.sync_copy(buf, outs[2 * j + 1].at[pl.ds(base, _BW)])


def _sc_gather(idx_s, idx_o, e_emb, abs_tables):
    out_type = []
    for j in range(_NT):
        d = STT if j == 0 else ABSD
        out_type += [jax.ShapeDtypeStruct((B, d), jnp.float32)] * 2
    mesh = plsc.VectorSubcoreMesh(core_axis_name="c", subcore_axis_name="s")
    k = functools.partial(
        pl.kernel, mesh=mesh, out_type=out_type,
        scratch_types=[
            pltpu.VMEM((_BW,), jnp.int32),
            pltpu.VMEM((_BW,), jnp.int32),
            pltpu.VMEM((_BW, STT), jnp.float32),
            pltpu.VMEM((_BW, ABSD), jnp.float32),
            pltpu.SemaphoreType.DMA,
        ],
    )(_sc_gather_fn)
    return k(idx_s, idx_o, e_emb, *abs_tables)


BLK = 256


def _dense_kernel(x_ref, es_ref, eo_ref, *rest):
    abs_refs = rest[:18]
    (r64_ref, rd_ref,
     fW1_ref, fb1_ref, fW2_ref, fb2_ref,
     pW1_ref, pb1_ref, pW2_ref, pb2_ref,
     aW1_ref, ab1_ref, aW2_ref, ab2_ref,
     out_ref) = rest[18:]
    x = x_ref[...]

    oh_r = _onehot(x[:, 1:2], 64)
    rr = _matmul(oh_r, r64_ref[...])
    es = es_ref[...]
    eo = eo_ref[...]

    t_d = x[:, 3:4].astype(jnp.float32)
    t_m = x[:, 4:5].astype(jnp.float32)
    t_y = x[:, 5:6].astype(jnp.float32)

    def abs_emb(which):
        out = jnp.zeros((BLK, ABSD), jnp.float32)
        for j, t in enumerate((t_d, t_m, t_y)):
            frq = abs_refs[2 * (3 * j + 0) + which][...]
            phi = abs_refs[2 * (3 * j + 1) + which][...]
            amp = abs_refs[2 * (3 * j + 2) + which][...]
            out = out + amp * jnp.sin(t * frq + phi)
        return out

    abs_s = abs_emb(0)
    abs_o = abs_emb(1)

    pid = pl.program_id(0)
    rows_b = jax.lax.broadcasted_iota(jnp.int32, (BLK, 64), 0) + pid * BLK
    cols_b = jax.lax.broadcasted_iota(jnp.int32, (BLK, 64), 1)
    oh16 = (jax.lax.div(rows_b, 16) == cols_b).astype(jnp.float32)

    rd = rd_ref[...]

    def mlp(d_part, e_rows, W1_ref, b1_ref, W2_ref, b2_ref):
        W1 = W1_ref[...]
        u = _matmul_t(d_part, W1[:, :REL0])
        h = _matmul(oh16, u) + _matmul_t(e_rows, W1[:, REL0:]) + b1_ref[...]
        h = jnp.maximum(h, 0.0)
        h = jnp.maximum(_matmul_t(h, W2_ref[...]) + b2_ref[...], 0.0)
        return h

    a_s = mlp(rd[:, 0 * REL0:1 * REL0], es, fW1_ref, fb1_ref, fW2_ref, fb2_ref)
    f_s = mlp(rd[:, 1 * REL0:2 * REL0], es, pW1_ref, pb1_ref, pW2_ref, pb2_ref)
    p_s = mlp(rd[:, 2 * REL0:3 * REL0], es, aW1_ref, ab1_ref, aW2_ref, ab2_ref)
    a_o = mlp(rd[:, 0 * REL0:1 * REL0], eo, fW1_ref, fb1_ref, fW2_ref, fb2_ref)
    f_o = mlp(rd[:, 1 * REL0:2 * REL0], eo, pW1_ref, pb1_ref, pW2_ref, pb2_ref)
    p_o = mlp(rd[:, 2 * REL0:3 * REL0], eo, aW1_ref, ab1_ref, aW2_ref, ab2_ref)

    cs = x[:, 6:6 + NREL]
    co = x[:, 6 + NREL:6 + 2 * NREL]
    val_cols = jax.lax.broadcasted_iota(jnp.int32, (BLK, NREL), 1)

    def hist(c):
        cnt = jnp.zeros((BLK, NREL), jnp.float32)
        for n in range(NREL):
            cnt = cnt + (c[:, n:n + 1] == val_cols).astype(jnp.float32)
        return cnt

    def sin_sum(c, f, p):
        cnt = hist(c)
        s1 = jnp.sin(f)
        c1 = jnp.cos(f)
        sv = jnp.zeros((BLK, RELL), jnp.float32)
        cv = jnp.ones((BLK, RELL), jnp.float32)
        acc_s = jnp.zeros((BLK, RELL), jnp.float32)
        acc_c = jnp.zeros((BLK, RELL), jnp.float32)
        for v in range(NREL):
            cv_v = cnt[:, v:v + 1]
            acc_s = acc_s + cv_v * sv
            acc_c = acc_c + cv_v * cv
            if v < NREL - 1:
                sv, cv = sv * c1 + cv * s1, cv * c1 - sv * s1
        return acc_s * jnp.cos(p) + acc_c * jnp.sin(p)

    rel_s = a_s * sin_sum(cs, f_s, p_s)
    rel_o = a_o * sin_sum(co, f_o, p_o)

    diff_e = es + rr[:, :STT] - eo
    diff_a = abs_s + rr[:, STT:STT + ABSD] - abs_o
    diff_r = rel_s + rr[:, STT + ABSD:] - rel_o
    total = (jnp.sum(jnp.abs(diff_e), axis=1, keepdims=True)
             + jnp.sum(jnp.abs(diff_a), axis=1, keepdims=True)
             + jnp.sum(jnp.abs(diff_r), axis=1, keepdims=True))
    out_ref[...] = GAMMA - total


def _onehot(idx_col, n):
    cols = jax.lax.broadcasted_iota(jnp.int32, (idx_col.shape[0], n), 1)
    return (idx_col == cols).astype(jnp.float32)


def _matmul_t(a, w):
    return jax.lax.dot_general(a, w, (((1,), (1,)), ((), ())),
                               preferred_element_type=jnp.float32)


def _matmul(a, b):
    return jax.lax.dot_general(a, b, (((1,), (0,)), ((), ())),
                               preferred_element_type=jnp.float32)


@jax.jit
def kernel(x, e_emb, r_emb,
           ad_frq, ad_phi, ad_amp,
           am_frq, am_phi, am_amp,
           ay_frq, ay_phi, ay_amp,
           rd_frq, rd_phi, rd_amp,
           fW1, fb1, fW2, fb2,
           pW1, pb1, pW2, pb2,
           aW1, ab1, aW2, ab2):
    idx_s = x[:, 0]
    idx_o = x[:, 2]
    abs_tables = [ad_frq, ad_phi, ad_amp, am_frq, am_phi, am_amp,
                  ay_frq, ay_phi, ay_amp]
    g = _sc_gather(idx_s, idx_o, e_emb, abs_tables)
    es, eo = g[0], g[1]
    abs_rows = list(g[2:])

    r64 = r_emb[:64]
    rd_cat = jnp.concatenate([rd_amp, rd_frq, rd_phi], axis=1)

    def rep(arr):
        return pl.BlockSpec(arr.shape, lambda i: (0, 0))

    operands = (x, es, eo, *abs_rows, r64, rd_cat,
                fW1, fb1.reshape(1, RELL), fW2, fb2.reshape(1, RELL),
                pW1, pb1.reshape(1, RELL), pW2, pb2.reshape(1, RELL),
                aW1, ab1.reshape(1, RELL), aW2, ab2.reshape(1, RELL))
    in_specs = [pl.BlockSpec((BLK, x.shape[1]), lambda i: (i, 0)),
                pl.BlockSpec((BLK, STT), lambda i: (i, 0)),
                pl.BlockSpec((BLK, STT), lambda i: (i, 0))]
    in_specs += [pl.BlockSpec((BLK, ABSD), lambda i: (i, 0))
                 for _ in range(18)]
    in_specs += [rep(a) for a in operands[21:]]
    out = pl.pallas_call(
        _dense_kernel,
        grid=(B // BLK,),
        in_specs=in_specs,
        out_specs=pl.BlockSpec((BLK, 1), lambda i: (i, 0)),
        out_shape=jax.ShapeDtypeStruct((B, 1), jnp.float32),
    )(*operands)
    return out
